# baseline (speedup 1.0000x reference)
import functools

import jax
import jax.numpy as jnp
from jax import lax
from jax.experimental import pallas as pl
from jax.experimental.pallas import tpu as pltpu
from jax.experimental.pallas import tpu_sc as plsc

K = 2
TM = 256
NC, NS = 2, 16
NW = NC * NS


def _dispatch_sc(x_hbm, p0_hbm, p1_hbm, xs_hbm, rows_v, i0_v, i1_v, sem, *,
                 cpw):
    wid = lax.axis_index("s") * NC + lax.axis_index("c")
    base = wid * cpw
    pltpu.sync_copy(x_hbm.at[pl.ds(base, cpw)], rows_v)
    pltpu.sync_copy(p0_hbm.at[pl.ds(base, cpw)], i0_v)
    pltpu.sync_copy(p1_hbm.at[pl.ds(base, cpw)], i1_v)
    c0 = pltpu.async_copy(rows_v, xs_hbm.at[i0_v], sem)
    c1 = pltpu.async_copy(rows_v, xs_hbm.at[i1_v], sem)
    c0.wait()
    c1.wait()


def _combine_sc(y_hbm, p0_hbm, p1_hbm, w1_hbm, w2_hbm, out_hbm,
                r0_v, r1_v, o_v, i0_v, i1_v, w1_v, w2_v, s0, s1, *,
                cpw, ch, h):
    wid = lax.axis_index("s") * NC + lax.axis_index("c")
    base = wid * cpw
    pltpu.sync_copy(p0_hbm.at[pl.ds(base, cpw)], i0_v)
    pltpu.sync_copy(p1_hbm.at[pl.ds(base, cpw)], i1_v)
    pltpu.sync_copy(w1_hbm.at[pl.ds(base, cpw)], w1_v)
    pltpu.sync_copy(w2_hbm.at[pl.ds(base, cpw)], w2_v)
    for c in range(cpw // ch):
        g0 = pltpu.async_copy(y_hbm.at[i0_v.at[pl.ds(c * ch, ch)]], r0_v, s0)
        g1 = pltpu.async_copy(y_hbm.at[i1_v.at[pl.ds(c * ch, ch)]], r1_v, s1)
        g0.wait()
        g1.wait()
        for i in range(ch):
            lane_i = jnp.full((16,), c * ch + i, jnp.int32)
            ws0 = plsc.load_gather(w1_v, [lane_i])
            ws1 = plsc.load_gather(w2_v, [lane_i])

            def body(j, carry, i=i, ws0=ws0, ws1=ws1):
                sl = pl.ds(j * 16, 16)
                o_v[i, sl] = r0_v[i, sl] * ws0 + r1_v[i, sl] * ws1
                return carry

            lax.fori_loop(0, h // 16, body, 0, unroll=8)
        pltpu.sync_copy(o_v, out_hbm.at[pl.ds(base + c * ch, ch)])


def _router_kernel(x_ref, gwt_ref, logits_ref, p0_ref, p1_ref, w1_ref, w2_ref,
                   xbf_ref, cnt_ref, off_ref, se_ref, st_ref, ns_ref):
    xf = x_ref[...]
    x = xf.astype(jnp.bfloat16)
    xbf_ref[...] = x
    logits = jnp.dot(x, gwt_ref[...].astype(jnp.bfloat16),
                     preferred_element_type=jnp.float32)
    logits_ref[...] = logits
    t, e = logits.shape
    lane = jax.lax.broadcasted_iota(jnp.int32, logits.shape, 1)
    l1 = jnp.max(logits, axis=1, keepdims=True)
    i1 = jnp.min(jnp.where(logits == l1, lane, e), axis=1, keepdims=True)
    masked = jnp.where(lane == i1, -jnp.inf, logits)
    l2 = jnp.max(masked, axis=1, keepdims=True)
    i2 = jnp.min(jnp.where(masked == l2, lane, e), axis=1, keepdims=True)
    r = jnp.exp(l2 - l1)
    w1_ref[...] = 1.0 / (1.0 + r)
    w2_ref[...] = 1.0 - 1.0 / (1.0 + r)

    oh1 = (lane == i1)
    oh2 = (lane == i2)
    oh_i = (oh1 | oh2).astype(jnp.int32)
    zrow = jnp.zeros((1, e), jnp.int32)
    c_excl = jnp.concatenate([zrow, oh_i[:-1]], axis=0)
    s = 1
    while s < t:
        c_excl = c_excl + jnp.concatenate(
            [jnp.zeros((min(s, t), e), jnp.int32), c_excl[:-s]], axis=0)
        s *= 2
    c_excl = c_excl.astype(jnp.float32)
    counts = jnp.sum(oh_i, axis=0, keepdims=True).astype(jnp.float32)
    er = jax.lax.broadcasted_iota(jnp.int32, (e, e), 0)
    ec = jax.lax.broadcasted_iota(jnp.int32, (e, e), 1)
    tri_e = (er < ec).astype(jnp.float32)
    off = jnp.dot(counts, tri_e, preferred_element_type=jnp.float32,
                  precision=jax.lax.Precision.HIGHEST)
    base = off + c_excl
    p0_ref[...] = jnp.sum(jnp.where(oh1, base, 0.0), axis=1,
                          keepdims=True).astype(jnp.int32)
    p1_ref[...] = jnp.sum(jnp.where(oh2, base, 0.0), axis=1,
                          keepdims=True).astype(jnp.int32)
    cnt_i = counts.astype(jnp.int32)
    off_i = off.astype(jnp.int32)
    ft = off_i // TM
    last = off_i + cnt_i - 1
    nt = jnp.where(cnt_i > 0, last // TM - ft + 1, 0)
    cnt_ref[...] = cnt_i
    off_ref[...] = off_i

    g = se_ref.shape[0]
    nt_f = nt.astype(jnp.float32)
    cume = jnp.dot(nt_f, tri_e, preferred_element_type=jnp.float32,
                   precision=jax.lax.Precision.HIGHEST)
    ns_ref[...] = jnp.sum(nt, axis=1, keepdims=True)
    srow = jax.lax.broadcasted_iota(jnp.int32, (g, e), 0)
    slane = jax.lax.broadcasted_iota(jnp.int32, (g, e), 1)
    cb = jnp.broadcast_to(cume.astype(jnp.int32), (g, e))
    se = jnp.sum((cb <= srow).astype(jnp.int32), axis=1, keepdims=True) - 1
    ohse = slane == se
    ft_b = jnp.broadcast_to(ft, (g, e))
    cume_b = jnp.broadcast_to(cume.astype(jnp.int32), (g, e))
    ftse = jnp.sum(jnp.where(ohse, ft_b, 0), axis=1, keepdims=True)
    cumese = jnp.sum(jnp.where(ohse, cume_b, 0), axis=1, keepdims=True)
    sidx = jax.lax.broadcasted_iota(jnp.int32, (g, 1), 0)
    ntiles = (t * K) // TM
    st = ftse + sidx - cumese
    st = jnp.clip(st, 0, ntiles - 1)
    se_ref[...] = se
    st_ref[...] = st


def _gmm_kernel(se_ref, st_ref, ns_ref, off_ref, cnt_ref,
                xs_ref, gp_ref, up_ref, dp_ref, y_ref, *, tm):
    s = pl.program_id(0)

    @pl.when(s < ns_ref[0])
    def _():
        e = se_ref[s]
        tile = st_ref[s]
        start = tile * tm
        off = off_ref[e]
        cnt = cnt_ref[e]
        rows = start + jax.lax.broadcasted_iota(jnp.int32, (tm, 1), 0)
        mask = (rows >= off) & (rows < off + cnt)
        xb = xs_ref[...]
        xg = jnp.dot(xb, gp_ref[0], preferred_element_type=jnp.float32)
        xu = jnp.dot(xb, up_ref[0], preferred_element_type=jnp.float32)
        h = (xg * jax.nn.sigmoid(xg)) * xu
        y = jnp.dot(h.astype(jnp.bfloat16), dp_ref[0],
                    preferred_element_type=jnp.float32)
        first = off <= start
        prev = jnp.where(first, jnp.zeros_like(y), y_ref[...])
        y_ref[...] = jnp.where(mask, y, prev)


def kernel(hidden_states, gate_w, gate_proj, up_proj, down_proj):
    B, S, H = hidden_states.shape
    E, _, FF = gate_proj.shape
    T = B * S
    A = T * K
    x = hidden_states.reshape(T, H)

    NT = A // TM
    G = (NT + E - 1 + 7) // 8 * 8
    (logits, p0, p1, w1, w2, xbf, cnt, off, se, st, ns) = pl.pallas_call(
        _router_kernel,
        out_shape=(
            jax.ShapeDtypeStruct((T, E), jnp.float32),
            jax.ShapeDtypeStruct((T, 1), jnp.int32),
            jax.ShapeDtypeStruct((T, 1), jnp.int32),
            jax.ShapeDtypeStruct((T, 1), jnp.float32),
            jax.ShapeDtypeStruct((T, 1), jnp.float32),
            jax.ShapeDtypeStruct((T, H), jnp.bfloat16),
            jax.ShapeDtypeStruct((1, E), jnp.int32),
            jax.ShapeDtypeStruct((1, E), jnp.int32),
            jax.ShapeDtypeStruct((G, 1), jnp.int32),
            jax.ShapeDtypeStruct((G, 1), jnp.int32),
            jax.ShapeDtypeStruct((1, 1), jnp.int32),
        ),
    )(x, gate_w.T)

    p0f = p0.reshape(T)
    p1f = p1.reshape(T)

    HW = H // 2
    x_i32 = jax.lax.bitcast_convert_type(xbf.reshape(T, HW, 2), jnp.int32)
    cpw = T // NW
    xs_i32 = pl.kernel(
        functools.partial(_dispatch_sc, cpw=cpw),
        out_type=jax.ShapeDtypeStruct((A, HW), jnp.int32),
        mesh=plsc.VectorSubcoreMesh(core_axis_name="c", subcore_axis_name="s"),
        compiler_params=pltpu.CompilerParams(needs_layout_passes=False),
        scratch_types=[
            pltpu.VMEM((cpw, HW), jnp.int32),
            pltpu.VMEM((cpw,), jnp.int32),
            pltpu.VMEM((cpw,), jnp.int32),
            pltpu.SemaphoreType.DMA,
        ],
    )(x_i32, p0f, p1f)
    xs = jax.lax.bitcast_convert_type(xs_i32, jnp.bfloat16).reshape(A, H)

    def x_idx(s, se, st, ns, off, cnt):
        return (st[s], 0)

    def w_idx(s, se, st, ns, off, cnt):
        return (se[s], 0, 0)

    grid_spec = pltpu.PrefetchScalarGridSpec(
        num_scalar_prefetch=5,
        grid=(G,),
        in_specs=[
            pl.BlockSpec((TM, H), x_idx),
            pl.BlockSpec((1, H, FF), w_idx),
            pl.BlockSpec((1, H, FF), w_idx),
            pl.BlockSpec((1, FF, H), w_idx),
        ],
        out_specs=pl.BlockSpec((TM, H), x_idx),
    )

    y = pl.pallas_call(
        functools.partial(_gmm_kernel, tm=TM),
        grid_spec=grid_spec,
        out_shape=jax.ShapeDtypeStruct((A, H), jnp.float32),
        compiler_params=pltpu.CompilerParams(
            dimension_semantics=("arbitrary",)),
    )(se.reshape(G), st.reshape(G), ns.reshape(1), off.reshape(E),
      cnt.reshape(E), xs, gate_proj.astype(jnp.bfloat16),
      up_proj.astype(jnp.bfloat16), down_proj.astype(jnp.bfloat16))

    out = y[p0f] * w1 + y[p1f] * w2
    return out.reshape(B, S, H), logits


# speedup vs baseline: 1.8277x; 1.8277x over previous
import functools

import jax
import jax.numpy as jnp
from jax import lax
from jax.experimental import pallas as pl
from jax.experimental.pallas import tpu as pltpu
from jax.experimental.pallas import tpu_sc as plsc

K = 2
TM = 256
NC, NS = 2, 16
NW = NC * NS


def _invperm_sc(ar_hbm, p0_hbm, p1_hbm, inv_hbm, v_v, i0_v, i1_v, sem, *,
                cpw):
    wid = lax.axis_index("s") * NC + lax.axis_index("c")
    base = wid * cpw
    pltpu.sync_copy(ar_hbm.at[pl.ds(base, cpw)], v_v)
    pltpu.sync_copy(p0_hbm.at[pl.ds(base, cpw)], i0_v)
    pltpu.sync_copy(p1_hbm.at[pl.ds(base, cpw)], i1_v)
    c0 = pltpu.async_copy(v_v, inv_hbm.at[i0_v], sem)
    c1 = pltpu.async_copy(v_v, inv_hbm.at[i1_v], sem)
    c0.wait()
    c1.wait()


def _router_kernel(x_ref, gwt_ref, logits_ref, p0_ref, p1_ref, w1_ref, w2_ref,
                   xbf_ref, cnt_ref, off_ref, se_ref, st_ref, ns_ref):
    xf = x_ref[...]
    x = xf.astype(jnp.bfloat16)
    xbf_ref[...] = x
    logits = jnp.dot(x, gwt_ref[...].astype(jnp.bfloat16),
                     preferred_element_type=jnp.float32)
    logits_ref[...] = logits
    t, e = logits.shape
    lane = jax.lax.broadcasted_iota(jnp.int32, logits.shape, 1)
    l1 = jnp.max(logits, axis=1, keepdims=True)
    i1 = jnp.min(jnp.where(logits == l1, lane, e), axis=1, keepdims=True)
    masked = jnp.where(lane == i1, -jnp.inf, logits)
    l2 = jnp.max(masked, axis=1, keepdims=True)
    i2 = jnp.min(jnp.where(masked == l2, lane, e), axis=1, keepdims=True)
    r = jnp.exp(l2 - l1)
    w1_ref[...] = 1.0 / (1.0 + r)
    w2_ref[...] = 1.0 - 1.0 / (1.0 + r)

    oh1 = (lane == i1)
    oh2 = (lane == i2)
    oh_i = (oh1 | oh2).astype(jnp.int32)
    zrow = jnp.zeros((1, e), jnp.int32)
    c_excl = jnp.concatenate([zrow, oh_i[:-1]], axis=0)
    s = 1
    while s < t:
        c_excl = c_excl + jnp.concatenate(
            [jnp.zeros((min(s, t), e), jnp.int32), c_excl[:-s]], axis=0)
        s *= 2
    c_excl = c_excl.astype(jnp.float32)
    counts = jnp.sum(oh_i, axis=0, keepdims=True).astype(jnp.float32)
    er = jax.lax.broadcasted_iota(jnp.int32, (e, e), 0)
    ec = jax.lax.broadcasted_iota(jnp.int32, (e, e), 1)
    tri_e = (er < ec).astype(jnp.float32)
    off = jnp.dot(counts, tri_e, preferred_element_type=jnp.float32,
                  precision=jax.lax.Precision.HIGHEST)
    base = off + c_excl
    p0_ref[...] = jnp.sum(jnp.where(oh1, base, 0.0), axis=1,
                          keepdims=True).astype(jnp.int32)
    p1_ref[...] = jnp.sum(jnp.where(oh2, base, 0.0), axis=1,
                          keepdims=True).astype(jnp.int32)
    cnt_i = counts.astype(jnp.int32)
    off_i = off.astype(jnp.int32)
    ft = off_i // TM
    last = off_i + cnt_i - 1
    nt = jnp.where(cnt_i > 0, last // TM - ft + 1, 0)
    cnt_ref[...] = cnt_i
    off_ref[...] = off_i

    g = se_ref.shape[0]
    nt_f = nt.astype(jnp.float32)
    cume = jnp.dot(nt_f, tri_e, preferred_element_type=jnp.float32,
                   precision=jax.lax.Precision.HIGHEST)
    ns_ref[...] = jnp.sum(nt, axis=1, keepdims=True)
    srow = jax.lax.broadcasted_iota(jnp.int32, (g, e), 0)
    slane = jax.lax.broadcasted_iota(jnp.int32, (g, e), 1)
    cb = jnp.broadcast_to(cume.astype(jnp.int32), (g, e))
    se = jnp.sum((cb <= srow).astype(jnp.int32), axis=1, keepdims=True) - 1
    ohse = slane == se
    ft_b = jnp.broadcast_to(ft, (g, e))
    cume_b = jnp.broadcast_to(cume.astype(jnp.int32), (g, e))
    ftse = jnp.sum(jnp.where(ohse, ft_b, 0), axis=1, keepdims=True)
    cumese = jnp.sum(jnp.where(ohse, cume_b, 0), axis=1, keepdims=True)
    sidx = jax.lax.broadcasted_iota(jnp.int32, (g, 1), 0)
    ntiles = (t * K) // TM
    st = ftse + sidx - cumese
    st = jnp.clip(st, 0, ntiles - 1)
    se_ref[...] = se
    st_ref[...] = st


def _gmm_kernel(se_ref, st_ref, ns_ref, off_ref, cnt_ref,
                xs_ref, gp_ref, up_ref, dp_ref, y_ref, *, tm):
    s = pl.program_id(0)

    @pl.when(s < ns_ref[0])
    def _():
        e = se_ref[s]
        tile = st_ref[s]
        start = tile * tm
        off = off_ref[e]
        cnt = cnt_ref[e]
        rows = start + jax.lax.broadcasted_iota(jnp.int32, (tm, 1), 0)
        mask = (rows >= off) & (rows < off + cnt)
        xb = xs_ref[...].astype(jnp.float32)
        xg = jnp.dot(xb, gp_ref[0], preferred_element_type=jnp.float32)
        xu = jnp.dot(xb, up_ref[0], preferred_element_type=jnp.float32)
        h = (xg * jax.nn.sigmoid(xg)) * xu
        y = jnp.dot(h, dp_ref[0], preferred_element_type=jnp.float32)
        first = off <= start
        prev = jnp.where(first, jnp.zeros_like(y), y_ref[...])
        y_ref[...] = jnp.where(mask, y, prev)


def kernel(hidden_states, gate_w, gate_proj, up_proj, down_proj):
    B, S, H = hidden_states.shape
    E, _, FF = gate_proj.shape
    T = B * S
    A = T * K
    x = hidden_states.reshape(T, H)

    NT = A // TM
    G = (NT + E - 1 + 7) // 8 * 8
    (logits, p0, p1, w1, w2, xbf, cnt, off, se, st, ns) = pl.pallas_call(
        _router_kernel,
        out_shape=(
            jax.ShapeDtypeStruct((T, E), jnp.float32),
            jax.ShapeDtypeStruct((T, 1), jnp.int32),
            jax.ShapeDtypeStruct((T, 1), jnp.int32),
            jax.ShapeDtypeStruct((T, 1), jnp.float32),
            jax.ShapeDtypeStruct((T, 1), jnp.float32),
            jax.ShapeDtypeStruct((T, H), jnp.bfloat16),
            jax.ShapeDtypeStruct((1, E), jnp.int32),
            jax.ShapeDtypeStruct((1, E), jnp.int32),
            jax.ShapeDtypeStruct((G, 1), jnp.int32),
            jax.ShapeDtypeStruct((G, 1), jnp.int32),
            jax.ShapeDtypeStruct((1, 1), jnp.int32),
        ),
    )(x, gate_w.T)

    p0f = p0.reshape(T)
    p1f = p1.reshape(T)

    cpw = T // NW
    inv = pl.kernel(
        functools.partial(_invperm_sc, cpw=cpw),
        out_type=jax.ShapeDtypeStruct((A,), jnp.int32),
        mesh=plsc.VectorSubcoreMesh(core_axis_name="c", subcore_axis_name="s"),
        compiler_params=pltpu.CompilerParams(needs_layout_passes=False),
        scratch_types=[
            pltpu.VMEM((cpw,), jnp.int32),
            pltpu.VMEM((cpw,), jnp.int32),
            pltpu.VMEM((cpw,), jnp.int32),
            pltpu.SemaphoreType.DMA,
        ],
    )(jnp.arange(T, dtype=jnp.int32), p0f, p1f)

    xs = xbf[inv]

    def x_idx(s, se, st, ns, off, cnt):
        return (st[s], 0)

    def w_idx(s, se, st, ns, off, cnt):
        return (se[s], 0, 0)

    grid_spec = pltpu.PrefetchScalarGridSpec(
        num_scalar_prefetch=5,
        grid=(G,),
        in_specs=[
            pl.BlockSpec((TM, H), x_idx),
            pl.BlockSpec((1, H, FF), w_idx),
            pl.BlockSpec((1, H, FF), w_idx),
            pl.BlockSpec((1, FF, H), w_idx),
        ],
        out_specs=pl.BlockSpec((TM, H), x_idx),
    )

    y = pl.pallas_call(
        functools.partial(_gmm_kernel, tm=TM),
        grid_spec=grid_spec,
        out_shape=jax.ShapeDtypeStruct((A, H), jnp.float32),
        compiler_params=pltpu.CompilerParams(
            dimension_semantics=("arbitrary",)),
    )(se.reshape(G), st.reshape(G), ns.reshape(1), off.reshape(E),
      cnt.reshape(E), xs, gate_proj, up_proj, down_proj)

    out = y[p0f] * w1 + y[p1f] * w2
    return out.reshape(B, S, H), logits
